# baseline (speedup 1.0000x reference)
import functools

import jax
import jax.numpy as jnp
from jax import lax
from jax.experimental import pallas as pl
from jax.experimental.pallas import tpu as pltpu
from jax.experimental.pallas import tpu_sc as plsc

_NC = 2
_NS = 16
_NW = _NC * _NS


def _precompute_body(x_ref, w1_ref, g_ref, b1_ref, t_ref, c0_ref):
    h = x_ref.shape[-1]
    xv = x_ref[...]
    t_ref[0, :, :] = jnp.dot(xv, w1_ref[0:h, :], preferred_element_type=jnp.float32)
    t_ref[1, :, :] = jnp.dot(xv, w1_ref[h:2 * h, :], preferred_element_type=jnp.float32)
    c0_ref[...] = (
        jnp.dot(g_ref[...], w1_ref[3 * h:4 * h, :], preferred_element_type=jnp.float32)
        + b1_ref[...]
    )


@functools.lru_cache(maxsize=None)
def _make_gather(es, h, chunk):
    per_w = es // _NW
    steps = per_w // chunk
    mesh = plsc.VectorSubcoreMesh(core_axis_name="c", subcore_axis_name="s")

    @functools.partial(
        pl.kernel,
        mesh=mesh,
        out_type=jax.ShapeDtypeStruct((es, h), jnp.float32),
        scratch_types=[
            pltpu.VMEM((per_w,), jnp.int32),
            pltpu.VMEM((per_w,), jnp.int32),
            pltpu.VMEM((chunk, h), jnp.float32),
            pltpu.VMEM((chunk, h), jnp.float32),
            pltpu.SemaphoreType.DMA,
            pltpu.SemaphoreType.DMA,
            pltpu.SemaphoreType.DMA,
        ],
    )
    def gather_kernel(t_hbm, tgt_hbm, src_hbm, out_hbm, idx_t, idx_s,
                      rows0, rows1, sem_t, sem_a, sem_s):
        wid = lax.axis_index("s") * _NC + lax.axis_index("c")
        base = wid * per_w
        rows = [rows0, rows1]

        pltpu.sync_copy(tgt_hbm.at[pl.ds(base, per_w)], idx_t)
        pltpu.sync_copy(src_hbm.at[pl.ds(base, per_w)], idx_s)

        def it(i):
            return idx_t.at[pl.ds(i * chunk, chunk)]

        def isrc(i):
            return idx_s.at[pl.ds(i * chunk, chunk)]

        h_t = [None] * steps
        h_s = [None] * steps
        h_t[0] = pltpu.async_copy(t_hbm.at[it(0)], rows[0], sem_t)
        for i in range(steps):
            b = i % 2
            h_t[i].wait()
            if i + 1 < steps:
                if i >= 1:
                    h_s[i - 1].wait()
                h_t[i + 1] = pltpu.async_copy(
                    t_hbm.at[it(i + 1)], rows[1 - b], sem_t)
            pltpu.async_copy(t_hbm.at[isrc(i)], rows[b], sem_a,
                             add=True).wait()
            h_s[i] = pltpu.async_copy(
                rows[b], out_hbm.at[pl.ds(base + i * chunk, chunk)], sem_s)
        if steps >= 2:
            h_s[steps - 2].wait()
        h_s[steps - 1].wait()

    return gather_kernel


def _mlp_body(g_ref, ea_ref, c_ref, c0_ref, w2_ref, b2_ref,
              gamma_ref, beta_ref, out_ref):
    pre = (
        g_ref[...]
        + jnp.dot(ea_ref[...], c_ref[...], preferred_element_type=jnp.float32)
        + c0_ref[...]
    )
    h1 = jnp.maximum(pre, 0.0)
    h2v = jnp.dot(h1, w2_ref[...], preferred_element_type=jnp.float32) + b2_ref[...]
    mean = jnp.mean(h2v, axis=-1, keepdims=True)
    d = h2v - mean
    var = jnp.mean(d * d, axis=-1, keepdims=True)
    out_ref[...] = d * lax.rsqrt(var + 1e-5) * gamma_ref[...] + beta_ref[...]


def _mlp_body_aliased(acc_ref, g_ref, ea_ref, c_ref, c0_ref, w2_ref,
                      b2_ref, gamma_ref, beta_ref, out_ref):
    del acc_ref
    _mlp_body(g_ref, ea_ref, c_ref, c0_ref, w2_ref, b2_ref,
              gamma_ref, beta_ref, out_ref)


def kernel(x, edge_index, edge_attr, global_attr, W1, b1, W2, b2, gamma, beta):
    n, h = x.shape
    e = edge_attr.shape[0]

    tbl3, c0 = pl.pallas_call(
        _precompute_body,
        out_shape=[
            jax.ShapeDtypeStruct((2, n, h), jnp.float32),
            jax.ShapeDtypeStruct((1, h), jnp.float32),
        ],
    )(x, W1, global_attr.reshape(1, h), b1.reshape(1, h))
    tbl = tbl3.reshape(2 * n, h)

    sizes = [25600, 51200, 76800, 76800, 89600]
    chunks = {25600: 400, 51200: 400, 76800: 400, 89600: 400}
    assert sum(sizes) == e
    ns = len(sizes)
    be = 6400
    offs = [sum(sizes[:k]) for k in range(ns)]

    tgt_all = edge_index[1].astype(jnp.int32)
    src_all = edge_index[0].astype(jnp.int32) + n
    g_list = [
        _make_gather(es_k, h, chunks[es_k])(
            tbl, tgt_all[o:o + es_k], src_all[o:o + es_k])
        for k, (es_k, o) in enumerate(zip(sizes, offs))
    ]

    weights = (W1[2 * h:3 * h, :], c0, W2, b2.reshape(1, h),
               gamma.reshape(1, h), beta.reshape(1, h))
    w_specs = [
        pl.BlockSpec((h, h), lambda i: (0, 0)),
        pl.BlockSpec((1, h), lambda i: (0, 0)),
        pl.BlockSpec((h, h), lambda i: (0, 0)),
        pl.BlockSpec((1, h), lambda i: (0, 0)),
        pl.BlockSpec((1, h), lambda i: (0, 0)),
        pl.BlockSpec((1, h), lambda i: (0, 0)),
    ]

    def stripe_specs(k):
        off = offs[k] // be
        return [
            pl.BlockSpec((be, h), lambda i: (i, 0)),
            pl.BlockSpec((be, h), lambda i, off=off: (off + i, 0)),
        ] + w_specs

    out_sds = jax.ShapeDtypeStruct((e, h), jnp.float32)

    out = pl.pallas_call(
        _mlp_body,
        grid=(sizes[0] // be,),
        in_specs=stripe_specs(0),
        out_specs=pl.BlockSpec((be, h), lambda i: (i, 0)),
        out_shape=out_sds,
    )(g_list[0], edge_attr, *weights)

    for k in range(1, ns):
        off = offs[k] // be
        out = pl.pallas_call(
            _mlp_body_aliased,
            grid=(sizes[k] // be,),
            in_specs=[pl.BlockSpec(memory_space=pltpu.MemorySpace.HBM)]
            + stripe_specs(k),
            out_specs=pl.BlockSpec((be, h),
                                   lambda i, off=off: (off + i, 0)),
            out_shape=out_sds,
            input_output_aliases={0: 0},
        )(out, g_list[k], edge_attr, *weights)
    return out


# speedup vs baseline: 1.0057x; 1.0057x over previous
import functools

import jax
import jax.numpy as jnp
from jax import lax
from jax.experimental import pallas as pl
from jax.experimental.pallas import tpu as pltpu
from jax.experimental.pallas import tpu_sc as plsc

_NC = 2
_NS = 16
_NW = _NC * _NS


def _precompute_body(x_ref, w1_ref, g_ref, b1_ref, t_ref, c0_ref):
    h = x_ref.shape[-1]
    xv = x_ref[...]
    t_ref[0, :, :] = jnp.dot(xv, w1_ref[0:h, :], preferred_element_type=jnp.float32)
    t_ref[1, :, :] = jnp.dot(xv, w1_ref[h:2 * h, :], preferred_element_type=jnp.float32)
    c0_ref[...] = (
        jnp.dot(g_ref[...], w1_ref[3 * h:4 * h, :], preferred_element_type=jnp.float32)
        + b1_ref[...]
    )


@functools.lru_cache(maxsize=None)
def _make_gather(es, h, chunk):
    per_w = es // _NW
    steps = per_w // chunk
    mesh = plsc.VectorSubcoreMesh(core_axis_name="c", subcore_axis_name="s")

    @functools.partial(
        pl.kernel,
        mesh=mesh,
        out_type=jax.ShapeDtypeStruct((es, h), jnp.float32),
        scratch_types=[
            pltpu.VMEM((per_w,), jnp.int32),
            pltpu.VMEM((per_w,), jnp.int32),
            pltpu.VMEM((chunk, h), jnp.float32),
            pltpu.VMEM((chunk, h), jnp.float32),
            pltpu.SemaphoreType.DMA,
            pltpu.SemaphoreType.DMA,
            pltpu.SemaphoreType.DMA,
        ],
    )
    def gather_kernel(t_hbm, tgt_hbm, src_hbm, out_hbm, idx_t, idx_s,
                      rows0, rows1, sem_t, sem_a, sem_s):
        wid = lax.axis_index("s") * _NC + lax.axis_index("c")
        base = wid * per_w
        rows = [rows0, rows1]

        pltpu.sync_copy(tgt_hbm.at[pl.ds(base, per_w)], idx_t)
        pltpu.sync_copy(src_hbm.at[pl.ds(base, per_w)], idx_s)

        def it(i):
            return idx_t.at[pl.ds(i * chunk, chunk)]

        def isrc(i):
            return idx_s.at[pl.ds(i * chunk, chunk)]

        h_t = [None] * steps
        h_s = [None] * steps
        h_t[0] = pltpu.async_copy(t_hbm.at[it(0)], rows[0], sem_t)
        for i in range(steps):
            b = i % 2
            h_t[i].wait()
            if i + 1 < steps:
                if i >= 1:
                    h_s[i - 1].wait()
                h_t[i + 1] = pltpu.async_copy(
                    t_hbm.at[it(i + 1)], rows[1 - b], sem_t)
            pltpu.async_copy(t_hbm.at[isrc(i)], rows[b], sem_a,
                             add=True).wait()
            h_s[i] = pltpu.async_copy(
                rows[b], out_hbm.at[pl.ds(base + i * chunk, chunk)], sem_s)
        if steps >= 2:
            h_s[steps - 2].wait()
        h_s[steps - 1].wait()

    return gather_kernel


def _mlp_body(g_ref, ea_ref, c_ref, c0_ref, w2_ref, b2_ref,
              gamma_ref, beta_ref, out_ref):
    pre = (
        g_ref[...]
        + jnp.dot(ea_ref[...], c_ref[...], preferred_element_type=jnp.float32)
        + c0_ref[...]
    )
    h1 = jnp.maximum(pre, 0.0)
    h2v = jnp.dot(h1, w2_ref[...], preferred_element_type=jnp.float32) + b2_ref[...]
    mean = jnp.mean(h2v, axis=-1, keepdims=True)
    d = h2v - mean
    var = jnp.mean(d * d, axis=-1, keepdims=True)
    out_ref[...] = d * lax.rsqrt(var + 1e-5) * gamma_ref[...] + beta_ref[...]


def _mlp_body_aliased(acc_ref, g_ref, ea_ref, c_ref, c0_ref, w2_ref,
                      b2_ref, gamma_ref, beta_ref, out_ref):
    del acc_ref
    _mlp_body(g_ref, ea_ref, c_ref, c0_ref, w2_ref, b2_ref,
              gamma_ref, beta_ref, out_ref)


def kernel(x, edge_index, edge_attr, global_attr, W1, b1, W2, b2, gamma, beta):
    n, h = x.shape
    e = edge_attr.shape[0]

    tbl3, c0 = pl.pallas_call(
        _precompute_body,
        out_shape=[
            jax.ShapeDtypeStruct((2, n, h), jnp.float32),
            jax.ShapeDtypeStruct((1, h), jnp.float32),
        ],
    )(x, W1, global_attr.reshape(1, h), b1.reshape(1, h))
    tbl = tbl3.reshape(2 * n, h)

    sizes = [38400, 51200, 76800, 76800, 76800]
    chunks = {38400: 400, 51200: 400, 76800: 400}
    assert sum(sizes) == e
    ns = len(sizes)
    be = 6400
    offs = [sum(sizes[:k]) for k in range(ns)]

    tgt_all = edge_index[1].astype(jnp.int32)
    src_all = edge_index[0].astype(jnp.int32) + n
    g_list = [
        _make_gather(es_k, h, chunks[es_k])(
            tbl, tgt_all[o:o + es_k], src_all[o:o + es_k])
        for k, (es_k, o) in enumerate(zip(sizes, offs))
    ]

    weights = (W1[2 * h:3 * h, :], c0, W2, b2.reshape(1, h),
               gamma.reshape(1, h), beta.reshape(1, h))
    w_specs = [
        pl.BlockSpec((h, h), lambda i: (0, 0)),
        pl.BlockSpec((1, h), lambda i: (0, 0)),
        pl.BlockSpec((h, h), lambda i: (0, 0)),
        pl.BlockSpec((1, h), lambda i: (0, 0)),
        pl.BlockSpec((1, h), lambda i: (0, 0)),
        pl.BlockSpec((1, h), lambda i: (0, 0)),
    ]

    def stripe_specs(k):
        off = offs[k] // be
        return [
            pl.BlockSpec((be, h), lambda i: (i, 0)),
            pl.BlockSpec((be, h), lambda i, off=off: (off + i, 0)),
        ] + w_specs

    out_sds = jax.ShapeDtypeStruct((e, h), jnp.float32)

    out = pl.pallas_call(
        _mlp_body,
        grid=(sizes[0] // be,),
        in_specs=stripe_specs(0),
        out_specs=pl.BlockSpec((be, h), lambda i: (i, 0)),
        out_shape=out_sds,
    )(g_list[0], edge_attr, *weights)

    for k in range(1, ns):
        off = offs[k] // be
        out = pl.pallas_call(
            _mlp_body_aliased,
            grid=(sizes[k] // be,),
            in_specs=[pl.BlockSpec(memory_space=pltpu.MemorySpace.HBM)]
            + stripe_specs(k),
            out_specs=pl.BlockSpec((be, h),
                                   lambda i, off=off: (off + i, 0)),
            out_shape=out_sds,
            input_output_aliases={0: 0},
        )(out, g_list[k], edge_attr, *weights)
    return out
